# baseline (speedup 1.0000x reference)
import functools

import jax
import jax.numpy as jnp
from jax import lax
from jax.experimental import pallas as pl
from jax.experimental.pallas import tpu as pltpu
from jax.experimental.pallas import tpu_sc as plsc

B, S, D = 4, 4096, 768
N = B * S
NW = 32
NS = 10240
NT = N - NS
ROWS_PER_W = NS // NW
C = 16
NCHUNK = ROWS_PER_W // C
NOB = 5
NGB = 5
PREF = 4
LANES = 16
DW = D // 2
VPAD = 128
TROWS = 640
MIDX_PER_W = 2 * ROWS_PER_W
TCB = 512

_mesh = plsc.VectorSubcoreMesh(core_axis_name="c", subcore_axis_name="s")


@functools.partial(
    pl.kernel,
    out_type=jax.ShapeDtypeStruct((N, D), jnp.float32),
    mesh=_mesh,
    scratch_types=[
        pltpu.VMEM((MIDX_PER_W,), jnp.int32),
        pltpu.VMEM((NOB, C, D), jnp.float32),
        pltpu.VMEM((NGB, 2 * C, DW), jnp.int32),
        pltpu.SemaphoreType.DMA((NOB,)),
        pltpu.SemaphoreType.DMA((NGB,)),
        pltpu.SemaphoreType.DMA((NOB,)),
    ],
)
def _emb_sum(x_hbm, midx_hbm, table_hbm, out_hbm,
             midx_v, obuf, gbuf, sem_x, sem_g, sem_o):
    wid = lax.axis_index("s") * 2 + lax.axis_index("c")
    base = wid * ROWS_PER_W
    pltpu.sync_copy(midx_hbm.at[pl.ds(wid * MIDX_PER_W, MIDX_PER_W)], midx_v)

    def issue_loads(g):
        so = lax.rem(g, NOB)
        sg = lax.rem(g, NGB)
        pltpu.async_copy(x_hbm.at[pl.ds(base + g * C, C)], obuf.at[so],
                         sem_x.at[so])
        pltpu.async_copy(table_hbm.at[midx_v.at[pl.ds(g * 2 * C, 2 * C)]],
                         gbuf.at[sg], sem_g.at[sg])

    def wait_loads(g):
        so = lax.rem(g, NOB)
        sg = lax.rem(g, NGB)
        pltpu.make_async_copy(x_hbm.at[pl.ds(base + g * C, C)], obuf.at[so],
                              sem_x.at[so]).wait()
        pltpu.make_async_copy(table_hbm.at[midx_v.at[pl.ds(g * 2 * C, 2 * C)]],
                              gbuf.at[sg], sem_g.at[sg]).wait()

    def wait_store(s):
        pltpu.make_async_copy(obuf.at[s], out_hbm.at[pl.ds(base, C)],
                              sem_o.at[s]).wait()

    for _i in range(PREF):
        issue_loads(jnp.int32(_i))

    hi_mask = jnp.int32(-65536)
    shift = jnp.int32(16)

    def chunk_body(g, carry):
        so = lax.rem(g, NOB)
        sg = lax.rem(g, NGB)
        wait_loads(g)

        def row_body(r, carry2):
            @plsc.parallel_loop(0, DW, LANES, unroll=6)
            def _vec(j):
                sl = pl.ds(j, LANES)
                wv = gbuf[sg, r, sl]
                wt = gbuf[sg, C + r, sl]
                lo = (lax.bitcast_convert_type(wv << shift, jnp.float32)
                      + lax.bitcast_convert_type(wt << shift, jnp.float32))
                hi = (lax.bitcast_convert_type(wv & hi_mask, jnp.float32)
                      + lax.bitcast_convert_type(wt & hi_mask, jnp.float32))
                plsc.addupdate(obuf.at[so, r, pl.ds(2 * j, LANES)], lo)
                plsc.addupdate(obuf.at[so, r, pl.ds(2 * j + LANES, LANES)], hi)
            return carry2

        lax.fori_loop(0, C, row_body, 0)

        pltpu.async_copy(obuf.at[so], out_hbm.at[pl.ds(base + g * C, C)],
                         sem_o.at[so])

        g2 = g + PREF

        @pl.when(g2 < NCHUNK)
        def _():
            @pl.when(g >= 1)
            def _():
                wait_store(lax.rem(g2, NOB))

            issue_loads(g2)

        return carry

    lax.fori_loop(0, NCHUNK, chunk_body, 0)
    for _i in range(min(NOB, NCHUNK)):
        wait_store(jnp.int32((NCHUNK - 1 - _i) % NOB))


def _tc_body(vidx_ref, tidx_ref, x_ref, tbl_ref, o_ref):
    vi = vidx_ref[0, 0, :]
    ti = tidx_ref[0, 0, :]
    iov = lax.broadcasted_iota(jnp.int32, (TCB, VPAD), 1)
    iot = lax.broadcasted_iota(jnp.int32, (TCB, TROWS - VPAD), 1)
    ohv = (iov == vi[:, None]).astype(jnp.bfloat16)
    oht = (iot == (ti[:, None] - VPAD)).astype(jnp.bfloat16)
    emb = (jnp.dot(ohv, tbl_ref[:VPAD], preferred_element_type=jnp.float32)
           + jnp.dot(oht, tbl_ref[VPAD:], preferred_element_type=jnp.float32))
    o_ref[...] = x_ref[...] + emb


_tc_gather = pl.pallas_call(
    _tc_body,
    grid=(NT // TCB,),
    in_specs=[
        pl.BlockSpec((1, 1, TCB), lambda i: (NS // TCB + i, 0, 0)),
        pl.BlockSpec((1, 1, TCB), lambda i: (NS // TCB + i, 0, 0)),
        pl.BlockSpec((TCB, D), lambda i: (NS // TCB + i, 0)),
        pl.BlockSpec((TROWS, D), lambda i: (0, 0)),
    ],
    out_specs=pl.BlockSpec((TCB, D), lambda i: (i, 0)),
    out_shape=jax.ShapeDtypeStruct((NT, D), jnp.float32),
)


def kernel(x, variable_seq, lead_time_seq, var_table, time_table):
    x2 = x.reshape(N, D)
    vidx = variable_seq.reshape(N).astype(jnp.int32)
    tidx = lead_time_seq.reshape(N).astype(jnp.int32) + VPAD
    midx = jnp.stack([vidx[:NS].reshape(-1, C), tidx[:NS].reshape(-1, C)],
                     axis=1).reshape(-1)
    table = jnp.concatenate(
        [var_table, jnp.zeros((VPAD - var_table.shape[0], D), jnp.float32),
         time_table,
         jnp.zeros((TROWS - VPAD - time_table.shape[0], D), jnp.float32)])
    tbl_bf = table.astype(jnp.bfloat16)
    bf = tbl_bf.reshape(TROWS, DW // LANES, 2, LANES)
    u16 = jax.lax.bitcast_convert_type(bf, jnp.uint16)
    packed = (u16[:, :, 0, :].astype(jnp.uint32)
              | (u16[:, :, 1, :].astype(jnp.uint32) << 16))
    packed = jax.lax.bitcast_convert_type(packed, jnp.int32).reshape(TROWS, DW)
    sc_out = _emb_sum(x2, midx, packed)
    tc_out = _tc_gather(vidx.reshape(N // TCB, 1, TCB),
                        tidx.reshape(N // TCB, 1, TCB), x2, tbl_bf)
    out = lax.dynamic_update_slice(sc_out, tc_out, (NS, 0))
    return out.reshape(B, S, D)


# speedup vs baseline: 1.0108x; 1.0108x over previous
import functools

import jax
import jax.numpy as jnp
from jax import lax
from jax.experimental import pallas as pl
from jax.experimental.pallas import tpu as pltpu
from jax.experimental.pallas import tpu_sc as plsc

B, S, D = 4, 4096, 768
N = B * S
NW = 32
ROWS_PER_W = N // NW
C = 16
NCHUNK = ROWS_PER_W // C
NOB = 5
NGB = 5
PREF = 4
LANES = 16
DW = D // 2
VOCAB = 100
TROWS = 640
MIDX_PER_W = 2 * ROWS_PER_W

_mesh = plsc.VectorSubcoreMesh(core_axis_name="c", subcore_axis_name="s")


@functools.partial(
    pl.kernel,
    out_type=jax.ShapeDtypeStruct((N, D), jnp.float32),
    mesh=_mesh,
    scratch_types=[
        pltpu.VMEM((MIDX_PER_W,), jnp.int32),
        pltpu.VMEM((NOB, C, D), jnp.float32),
        pltpu.VMEM((NGB, 2 * C, DW), jnp.int32),
        pltpu.SemaphoreType.DMA((NOB,)),
        pltpu.SemaphoreType.DMA((NGB,)),
        pltpu.SemaphoreType.DMA((NOB,)),
    ],
)
def _emb_sum(x_hbm, midx_hbm, table_hbm, out_hbm,
             midx_v, obuf, gbuf, sem_x, sem_g, sem_o):
    wid = lax.axis_index("s") * 2 + lax.axis_index("c")
    base = wid * ROWS_PER_W
    pltpu.sync_copy(midx_hbm.at[pl.ds(wid * MIDX_PER_W, MIDX_PER_W)], midx_v)

    def issue_loads(g):
        so = lax.rem(g, NOB)
        sg = lax.rem(g, NGB)
        pltpu.async_copy(x_hbm.at[pl.ds(base + g * C, C)], obuf.at[so],
                         sem_x.at[so])
        pltpu.async_copy(table_hbm.at[midx_v.at[pl.ds(g * 2 * C, 2 * C)]],
                         gbuf.at[sg], sem_g.at[sg])

    def wait_loads(g):
        so = lax.rem(g, NOB)
        sg = lax.rem(g, NGB)
        pltpu.make_async_copy(x_hbm.at[pl.ds(base + g * C, C)], obuf.at[so],
                              sem_x.at[so]).wait()
        pltpu.make_async_copy(table_hbm.at[midx_v.at[pl.ds(g * 2 * C, 2 * C)]],
                              gbuf.at[sg], sem_g.at[sg]).wait()

    def wait_store(s):
        pltpu.make_async_copy(obuf.at[s], out_hbm.at[pl.ds(base, C)],
                              sem_o.at[s]).wait()

    for _i in range(PREF):
        issue_loads(jnp.int32(_i))

    hi_mask = jnp.int32(-65536)
    shift = jnp.int32(16)

    def chunk_body(g, carry):
        so = lax.rem(g, NOB)
        sg = lax.rem(g, NGB)
        wait_loads(g)

        @plsc.parallel_loop(0, C, 1, unroll=1)
        def _row(r):
            for jb in range(DW // LANES):
                sl = pl.ds(jb * LANES, LANES)
                wv = gbuf[sg, r, sl]
                wt = gbuf[sg, C + r, sl]
                lo = (lax.bitcast_convert_type(wv << shift, jnp.float32)
                      + lax.bitcast_convert_type(wt << shift, jnp.float32))
                hi = (lax.bitcast_convert_type(wv & hi_mask, jnp.float32)
                      + lax.bitcast_convert_type(wt & hi_mask, jnp.float32))
                plsc.addupdate(obuf.at[so, r, pl.ds(2 * jb * LANES, LANES)], lo)
                plsc.addupdate(obuf.at[so, r, pl.ds((2 * jb + 1) * LANES, LANES)],
                               hi)

        pltpu.async_copy(obuf.at[so], out_hbm.at[pl.ds(base + g * C, C)],
                         sem_o.at[so])

        g2 = g + PREF

        @pl.when(g2 < NCHUNK)
        def _():
            @pl.when(g >= 1)
            def _():
                wait_store(lax.rem(g2, NOB))

            issue_loads(g2)

        return carry

    lax.fori_loop(0, NCHUNK, chunk_body, 0)
    for _i in range(min(NOB, NCHUNK)):
        wait_store(jnp.int32((NCHUNK - 1 - _i) % NOB))


def kernel(x, variable_seq, lead_time_seq, var_table, time_table):
    x2 = x.reshape(N, D)
    vidx = variable_seq.reshape(N).astype(jnp.int32)
    tidx = lead_time_seq.reshape(N).astype(jnp.int32) + VOCAB
    midx = jnp.stack([vidx.reshape(-1, C), tidx.reshape(-1, C)],
                     axis=1).reshape(-1)
    table = jnp.concatenate(
        [var_table, time_table,
         jnp.zeros((TROWS - VOCAB - time_table.shape[0], D), jnp.float32)])
    bf = table.astype(jnp.bfloat16).reshape(TROWS, DW // LANES, 2, LANES)
    u16 = jax.lax.bitcast_convert_type(bf, jnp.uint16)
    packed = (u16[:, :, 0, :].astype(jnp.uint32)
              | (u16[:, :, 1, :].astype(jnp.uint32) << 16))
    packed = jax.lax.bitcast_convert_type(packed, jnp.int32).reshape(TROWS, DW)
    out = _emb_sum(x2, midx, packed)
    return out.reshape(B, S, D)
